# baseline (speedup 1.0000x reference)
import functools

import jax
import jax.numpy as jnp
from jax import lax
from jax.experimental import pallas as pl
from jax.experimental.pallas import tpu as pltpu
from jax.experimental.pallas import tpu_sc as plsc

M_ITEM = 1000000
DIM = 64
B = 1024
HIST = 200
N_NEG = 100

L = 16
NGRP = 7
NPAD = NGRP * L
ROWS = HIST + NPAD + 8
TGT_ROW = HIST + NPAD

NC = 2
NS = 16
NW = NC * NS
BPW = B // NW


def _shuffle(v, idx16):
    return lax.gather(
        v, idx16[:, None],
        lax.GatherDimensionNumbers(offset_dims=(), collapsed_slice_dims=(0,),
                                   start_index_map=(0,)),
        (1,), mode=lax.GatherScatterMode.PROMISE_IN_BOUNDS)


def _lanesum(v, lanes):
    for k in (8, 4, 2, 1):
        v = v + _shuffle(v, lanes ^ k)
    return v


def _sc_body(table, idx, scal_out, nd_out, nn_out,
             idx_v, rows_v, scal_v, nd_v, nn_v, gsem):
    wid = lax.axis_index("s") * NC + lax.axis_index("c")
    base = wid * BPW

    pltpu.sync_copy(idx.at[pl.ds(base * ROWS, BPW * ROWS)], idx_v)

    lanes = jnp.arange(L, dtype=jnp.int32)

    def batch_body(bi, carry):
        off = bi * ROWS
        c0 = pltpu.async_copy(table.at[idx_v.at[pl.ds(off, 128)]],
                              rows_v.at[pl.ds(0, 128)], gsem)
        c1 = pltpu.async_copy(table.at[idx_v.at[pl.ds(off + 128, 128)]],
                              rows_v.at[pl.ds(128, 128)], gsem)
        c2 = pltpu.async_copy(table.at[idx_v.at[pl.ds(off + 256, 64)]],
                              rows_v.at[pl.ds(256, 64)], gsem)
        c0.wait()
        c1.wait()
        c2.wait()

        t0 = rows_v[TGT_ROW, pl.ds(0, L)]
        t1 = rows_v[TGT_ROW, pl.ds(L, L)]
        t2 = rows_v[TGT_ROW, pl.ds(2 * L, L)]
        t3 = rows_v[TGT_ROW, pl.ds(3 * L, L)]

        zero = jnp.zeros((L,), jnp.float32)

        def hist_body(r, acc):
            a0, a1, a2, a3 = acc
            return (a0 + rows_v[r, pl.ds(0, L)],
                    a1 + rows_v[r, pl.ds(L, L)],
                    a2 + rows_v[r, pl.ds(2 * L, L)],
                    a3 + rows_v[r, pl.ds(3 * L, L)])

        a0, a1, a2, a3 = lax.fori_loop(0, HIST, hist_body,
                                       (zero, zero, zero, zero))

        pos = _lanesum(a0 * t0 + a1 * t1 + a2 * t2 + a3 * t3, lanes)
        m2 = _lanesum(a0 * a0 + a1 * a1 + a2 * a2 + a3 * a3, lanes)
        tt = _lanesum(t0 * t0 + t1 * t1 + t2 * t2 + t3 * t3, lanes)
        zero = jnp.zeros((L,), jnp.float32)
        sv = jnp.where(lanes == 0, pos,
                       jnp.where(lanes == 1, m2,
                                 jnp.where(lanes == 2, tt, zero)))
        scal_v[bi, pl.ds(0, L)] = sv

        for g in range(NGRP):
            def neg_body(j, carry):
                dv, nv = carry
                r = HIST + g * L + j
                n0 = rows_v[r, pl.ds(0, L)]
                n1 = rows_v[r, pl.ds(L, L)]
                n2 = rows_v[r, pl.ds(2 * L, L)]
                n3 = rows_v[r, pl.ds(3 * L, L)]
                d = _lanesum(n0 * t0 + n1 * t1 + n2 * t2 + n3 * t3, lanes)
                nn = _lanesum(n0 * n0 + n1 * n1 + n2 * n2 + n3 * n3, lanes)
                dv = jnp.where(lanes == j, d, dv)
                nv = jnp.where(lanes == j, nn, nv)
                return dv, nv

            dv, nv = lax.fori_loop(0, L, neg_body, (zero, zero))
            nd_v[bi, pl.ds(g * L, L)] = dv
            nn_v[bi, pl.ds(g * L, L)] = nv
        return carry

    lax.fori_loop(0, BPW, batch_body, 0)

    pltpu.sync_copy(scal_v, scal_out.at[pl.ds(base, BPW)])
    pltpu.sync_copy(nd_v, nd_out.at[pl.ds(base, BPW)])
    pltpu.sync_copy(nn_v, nn_out.at[pl.ds(base, BPW)])


def _sc_gather_reduce(table, idx_flat):
    mesh = plsc.VectorSubcoreMesh(core_axis_name="c", subcore_axis_name="s")
    f = pl.kernel(
        _sc_body,
        mesh=mesh,
        out_type=(
            jax.ShapeDtypeStruct((B, L), jnp.float32),
            jax.ShapeDtypeStruct((B, NPAD), jnp.float32),
            jax.ShapeDtypeStruct((B, NPAD), jnp.float32),
        ),
        scratch_types=[
            pltpu.VMEM((BPW * ROWS,), jnp.int32),
            pltpu.VMEM((ROWS, DIM), jnp.float32),
            pltpu.VMEM((BPW, L), jnp.float32),
            pltpu.VMEM((BPW, NPAD), jnp.float32),
            pltpu.VMEM((BPW, NPAD), jnp.float32),
            pltpu.SemaphoreType.DMA,
        ],
        compiler_params=pltpu.CompilerParams(use_tc_tiling_on_sc=False),
    )
    return f(table, idx_flat)


def _tc_body(scal_ref, nd_ref, nn_ref, out_ref):
    eps = 1e-8
    pos_dot = scal_ref[:, 0:1]
    m2 = scal_ref[:, 1:2]
    tt = scal_ref[:, 2:3]
    na = jnp.maximum(jnp.sqrt(tt), eps)
    nb = jnp.maximum(jnp.sqrt(m2) * (1.0 / HIST), eps)
    pos_sim = (pos_dot * (1.0 / HIST)) / (na * nb)

    nd = nd_ref[:]
    nn = nn_ref[:]
    nbn = jnp.maximum(jnp.sqrt(nn), eps)
    neg_sim = nd / (na * nbn)
    col = lax.broadcasted_iota(jnp.int32, (B, NPAD), 1)
    neg_sim = jnp.where(col < N_NEG, neg_sim, -1e30)

    logits = jnp.concatenate([pos_sim, neg_sim], axis=1)
    mx = jnp.max(logits, axis=1, keepdims=True)
    lse = mx + jnp.log(jnp.sum(jnp.exp(logits - mx), axis=1, keepdims=True))
    logp0 = pos_sim - lse
    out_ref[...] = jnp.reshape(-jnp.sum(logp0) * (1.0 / B), (1, 1))


def _tc_loss(scal, nd, nn):
    return pl.pallas_call(
        _tc_body,
        out_shape=jax.ShapeDtypeStruct((1, 1), jnp.float32),
    )(scal, nd, nn)


def kernel(table, train_idx, neg_idx, target_idx):
    npad = jnp.zeros((B, NPAD - N_NEG), jnp.int32)
    tail = jnp.zeros((B, ROWS - HIST - NPAD - 1), jnp.int32)
    idx_flat = jnp.concatenate(
        [train_idx.astype(jnp.int32),
         neg_idx.astype(jnp.int32),
         npad,
         target_idx.astype(jnp.int32)[:, None],
         tail], axis=1).reshape(-1)
    scal, nd, nn = _sc_gather_reduce(table, idx_flat)
    loss = _tc_loss(scal, nd, nn)
    return jnp.reshape(loss, ())
